# baseline (speedup 1.0000x reference)
import jax
import jax.numpy as jnp
from jax import lax
from jax.experimental import pallas as pl
from jax.experimental.pallas import tpu as pltpu
from jax.experimental.pallas import tpu_sc as plsc

NC = 2
NS = 16
NW = NC * NS
LANES = 16
K = 4

N_NODES = 100000
N_LINKS = 199350
N_PER_W = 3136
N_TAIL = N_NODES - (NW - 1) * N_PER_W
IDX_PER_W = K * N_PER_W
IDX_TAIL = K * N_TAIL
CHUNKS = N_PER_W // LANES
STAGE = 12464
STAGE_TAIL = N_LINKS - (NS - 1) * STAGE


def _sc_body(array_hbm, idx_hbm, dirs_hbm, status_hbm, area_hbm, out_hbm,
             table_sp, idx_v, gath_v, dirs_v, status_v, area_v, out_v, sem):
    cid = lax.axis_index("c")
    sid = lax.axis_index("s")
    wid = sid * NC + cid
    is_tail = wid == NW - 1

    @pl.when(sid < NS - 1)
    def _():
        off = pl.multiple_of(sid * STAGE, 8)
        pltpu.sync_copy(array_hbm.at[pl.ds(off, STAGE)],
                        gath_v.at[pl.ds(0, STAGE)])
        pltpu.sync_copy(gath_v.at[pl.ds(0, STAGE)],
                        table_sp.at[pl.ds(off, STAGE)])

    @pl.when(sid == NS - 1)
    def _():
        off = (NS - 1) * STAGE
        pltpu.sync_copy(array_hbm.at[pl.ds(off, STAGE_TAIL)],
                        gath_v.at[pl.ds(0, STAGE_TAIL)])
        pltpu.sync_copy(gath_v.at[pl.ds(0, STAGE_TAIL)],
                        table_sp.at[pl.ds(off, STAGE_TAIL)])

    nbase = pl.multiple_of(wid * N_PER_W, 8)
    ibase = pl.multiple_of(wid * IDX_PER_W, 8)

    @pl.when(jnp.logical_not(is_tail))
    def _():
        pltpu.sync_copy(idx_hbm.at[pl.ds(ibase, IDX_PER_W)], idx_v)
        pltpu.sync_copy(dirs_hbm.at[pl.ds(ibase, IDX_PER_W)], dirs_v)
        pltpu.sync_copy(status_hbm.at[pl.ds(nbase, N_PER_W)], status_v)
        pltpu.sync_copy(area_hbm.at[pl.ds(nbase, N_PER_W)], area_v)

    @pl.when(is_tail)
    def _():
        pltpu.sync_copy(idx_hbm.at[pl.ds(ibase, IDX_TAIL)],
                        idx_v.at[pl.ds(0, IDX_TAIL)])
        pltpu.sync_copy(dirs_hbm.at[pl.ds(ibase, IDX_TAIL)],
                        dirs_v.at[pl.ds(0, IDX_TAIL)])
        pltpu.sync_copy(status_hbm.at[pl.ds(nbase, N_TAIL)],
                        status_v.at[pl.ds(0, N_TAIL)])
        pltpu.sync_copy(area_hbm.at[pl.ds(nbase, N_TAIL)],
                        area_v.at[pl.ds(0, N_TAIL)])

        def zfill(i, carry):
            idx_v[pl.ds(IDX_TAIL + i * LANES, LANES)] = jnp.zeros(
                (LANES,), jnp.int32)
            return carry
        lax.fori_loop(0, (IDX_PER_W - IDX_TAIL) // LANES, zfill, 0)

    plsc.subcore_barrier()

    pltpu.async_copy(table_sp.at[idx_v], gath_v, sem).wait()

    lane4 = lax.iota(jnp.int32, LANES) * K

    def chunk(c, carry):
        off = c * LANES
        acc = jnp.zeros((LANES,), jnp.float32)
        for j in range(K):
            pos = lane4 + (off * K + j)
            g = plsc.load_gather(gath_v, [pos])
            d = plsc.load_gather(dirs_v, [pos])
            acc = acc + d.astype(jnp.float32) * g
        st = status_v[pl.ds(off, LANES)]
        ar = area_v[pl.ds(off, LANES)]
        out_v[pl.ds(off, LANES)] = jnp.where(st == 0, acc / ar, 0.0)
        return carry

    lax.fori_loop(0, CHUNKS, chunk, 0)

    @pl.when(jnp.logical_not(is_tail))
    def _():
        pltpu.sync_copy(out_v, out_hbm.at[pl.ds(nbase, N_PER_W)])

    @pl.when(is_tail)
    def _():
        pltpu.sync_copy(out_v.at[pl.ds(0, N_TAIL)],
                        out_hbm.at[pl.ds(nbase, N_TAIL)])


@jax.jit
def _flux_div_sc(array, links_flat, dirs_flat, status, area):
    mesh = plsc.VectorSubcoreMesh(core_axis_name="c", subcore_axis_name="s")
    run = pl.kernel(
        _sc_body,
        out_type=jax.ShapeDtypeStruct((N_NODES,), jnp.float32),
        mesh=mesh,
        scratch_types=[
            pltpu.VMEM_SHARED((N_LINKS,), jnp.float32),
            pltpu.VMEM((IDX_PER_W,), jnp.int32),
            pltpu.VMEM((IDX_PER_W,), jnp.float32),
            pltpu.VMEM((IDX_PER_W,), jnp.int32),
            pltpu.VMEM((N_PER_W,), jnp.int32),
            pltpu.VMEM((N_PER_W,), jnp.float32),
            pltpu.VMEM((N_PER_W,), jnp.float32),
            pltpu.SemaphoreType.DMA,
        ],
        compiler_params=pltpu.CompilerParams(needs_layout_passes=False),
    )
    return run(array, links_flat, dirs_flat, status, area)


def kernel(array, cell_area_at_node, links_at_node, link_dirs_at_node, status_at_node):
    return _flux_div_sc(
        array,
        links_at_node.reshape(K * N_NODES),
        link_dirs_at_node.reshape(K * N_NODES),
        status_at_node,
        cell_area_at_node,
    )


# speedup vs baseline: 4.0418x; 4.0418x over previous
import jax
import jax.numpy as jnp
from jax import lax
from jax.experimental import pallas as pl
from jax.experimental.pallas import tpu as pltpu
from jax.experimental.pallas import tpu_sc as plsc

NC = 2
NS = 16
NW = NC * NS
LANES = 16
K = 4

N_NODES = 100000
N_LINKS = 199350
N_PER_W = 3136
N_TAIL = N_NODES - (NW - 1) * N_PER_W
IDX_PER_W = K * N_PER_W
CHUNKS = N_PER_W // LANES
TAIL_CHUNKS = N_TAIL // LANES
STAGE = 12464
STAGE_TAIL = N_LINKS - (NS - 1) * STAGE


def _sc_body(array_hbm, idxT_hbm, dirsT_hbm, status_hbm, area_hbm, out_hbm,
             table_sp, idx_v, gath_v, dirs_v, status_v, area_v, out_v, sem):
    cid = lax.axis_index("c")
    sid = lax.axis_index("s")
    wid = sid * NC + cid
    is_tail = wid == NW - 1

    @pl.when(sid < NS - 1)
    def _():
        off = pl.multiple_of(sid * STAGE, 8)
        pltpu.sync_copy(array_hbm.at[pl.ds(off, STAGE)],
                        gath_v.at[pl.ds(0, STAGE)])
        pltpu.sync_copy(gath_v.at[pl.ds(0, STAGE)],
                        table_sp.at[pl.ds(off, STAGE)])

    @pl.when(sid == NS - 1)
    def _():
        off = (NS - 1) * STAGE
        pltpu.sync_copy(array_hbm.at[pl.ds(off, STAGE_TAIL)],
                        gath_v.at[pl.ds(0, STAGE_TAIL)])
        pltpu.sync_copy(gath_v.at[pl.ds(0, STAGE_TAIL)],
                        table_sp.at[pl.ds(off, STAGE_TAIL)])

    nbase = pl.multiple_of(wid * N_PER_W, 8)

    @pl.when(jnp.logical_not(is_tail))
    def _():
        for j in range(K):
            pltpu.sync_copy(idxT_hbm.at[pl.ds(j * N_NODES + nbase, N_PER_W)],
                            idx_v.at[pl.ds(j * N_PER_W, N_PER_W)])
            pltpu.sync_copy(dirsT_hbm.at[pl.ds(j * N_NODES + nbase, N_PER_W)],
                            dirs_v.at[pl.ds(j * N_PER_W, N_PER_W)])
        pltpu.sync_copy(status_hbm.at[pl.ds(nbase, N_PER_W)], status_v)
        pltpu.sync_copy(area_hbm.at[pl.ds(nbase, N_PER_W)], area_v)

    @pl.when(is_tail)
    def _():
        for j in range(K):
            pltpu.sync_copy(idxT_hbm.at[pl.ds(j * N_NODES + nbase, N_TAIL)],
                            idx_v.at[pl.ds(j * N_PER_W, N_TAIL)])
            pltpu.sync_copy(dirsT_hbm.at[pl.ds(j * N_NODES + nbase, N_TAIL)],
                            dirs_v.at[pl.ds(j * N_PER_W, N_TAIL)])
        pltpu.sync_copy(status_hbm.at[pl.ds(nbase, N_TAIL)],
                        status_v.at[pl.ds(0, N_TAIL)])
        pltpu.sync_copy(area_hbm.at[pl.ds(nbase, N_TAIL)],
                        area_v.at[pl.ds(0, N_TAIL)])

        def zfill(i, carry):
            j, c = i // (CHUNKS - TAIL_CHUNKS), i % (CHUNKS - TAIL_CHUNKS)
            idx_v[pl.ds(j * N_PER_W + N_TAIL + c * LANES, LANES)] = (
                jnp.zeros((LANES,), jnp.int32))
            return carry
        lax.fori_loop(0, K * (CHUNKS - TAIL_CHUNKS), zfill, 0)

    plsc.subcore_barrier()

    pltpu.async_copy(table_sp.at[idx_v], gath_v, sem).wait()

    def chunk(c, carry):
        off = c * LANES
        acc = jnp.zeros((LANES,), jnp.float32)
        for j in range(K):
            g = gath_v[pl.ds(j * N_PER_W + off, LANES)]
            d = dirs_v[pl.ds(j * N_PER_W + off, LANES)]
            acc = acc + d * g
        st = status_v[pl.ds(off, LANES)]
        ar = area_v[pl.ds(off, LANES)]
        out_v[pl.ds(off, LANES)] = jnp.where(st == 0, acc / ar, 0.0)
        return carry

    lax.fori_loop(0, CHUNKS, chunk, 0)

    @pl.when(jnp.logical_not(is_tail))
    def _():
        pltpu.sync_copy(out_v, out_hbm.at[pl.ds(nbase, N_PER_W)])

    @pl.when(is_tail)
    def _():
        pltpu.sync_copy(out_v.at[pl.ds(0, N_TAIL)],
                        out_hbm.at[pl.ds(nbase, N_TAIL)])


@jax.jit
def _flux_div_sc(array, links_T, dirs_T, status, area):
    mesh = plsc.VectorSubcoreMesh(core_axis_name="c", subcore_axis_name="s")
    run = pl.kernel(
        _sc_body,
        out_type=jax.ShapeDtypeStruct((N_NODES,), jnp.float32),
        mesh=mesh,
        scratch_types=[
            pltpu.VMEM_SHARED((N_LINKS,), jnp.float32),
            pltpu.VMEM((IDX_PER_W,), jnp.int32),
            pltpu.VMEM((IDX_PER_W,), jnp.float32),
            pltpu.VMEM((IDX_PER_W,), jnp.float32),
            pltpu.VMEM((N_PER_W,), jnp.int32),
            pltpu.VMEM((N_PER_W,), jnp.float32),
            pltpu.VMEM((N_PER_W,), jnp.float32),
            pltpu.SemaphoreType.DMA,
        ],
        compiler_params=pltpu.CompilerParams(needs_layout_passes=False),
    )
    return run(array, links_T, dirs_T, status, area)


def kernel(array, cell_area_at_node, links_at_node, link_dirs_at_node, status_at_node):
    return _flux_div_sc(
        array,
        jnp.swapaxes(links_at_node, 0, 1).reshape(K * N_NODES),
        jnp.swapaxes(link_dirs_at_node, 0, 1).astype(jnp.float32).reshape(K * N_NODES),
        status_at_node,
        cell_area_at_node,
    )


# speedup vs baseline: 4.7789x; 1.1824x over previous
import jax
import jax.numpy as jnp
from jax import lax
from jax.experimental import pallas as pl
from jax.experimental.pallas import tpu as pltpu
from jax.experimental.pallas import tpu_sc as plsc

NC = 2
NS = 16
NW = NC * NS
LANES = 16
K = 4

N_NODES = 100000
N_LINKS = 199350
N_PER_W = 3136
N_TAIL = N_NODES - (NW - 1) * N_PER_W
IDX_PER_W = K * N_PER_W
CHUNKS = N_PER_W // LANES
TAIL_CHUNKS = N_TAIL // LANES
STAGE = 12464
STAGE_TAIL = N_LINKS - (NS - 1) * STAGE


def _sc_body(array_hbm, idxT_hbm, dirsT_hbm, status_hbm, area_hbm, out_hbm,
             table_sp, idx_v, gath_v, dirs_v, status_v, area_v, out_v, sem):
    cid = lax.axis_index("c")
    sid = lax.axis_index("s")
    wid = sid * NC + cid
    is_tail = wid == NW - 1

    nbase = pl.multiple_of(wid * N_PER_W, 8)
    stage_off = pl.multiple_of(sid * STAGE, 8)

    def stage_pairs(n):
        prs = [(array_hbm.at[pl.ds(stage_off, STAGE)],
                gath_v.at[pl.ds(0, STAGE)])] if n == STAGE else [
              (array_hbm.at[pl.ds((NS - 1) * STAGE, STAGE_TAIL)],
               gath_v.at[pl.ds(0, STAGE_TAIL)])]
        return prs

    def chunk_pairs(cnt):
        prs = []
        for j in range(K):
            prs.append((idxT_hbm.at[pl.ds(j * N_NODES + nbase, cnt)],
                        idx_v.at[pl.ds(j * N_PER_W, cnt)]))
            prs.append((dirsT_hbm.at[pl.ds(j * N_NODES + nbase, cnt)],
                        dirs_v.at[pl.ds(j * N_PER_W, cnt)]))
        prs.append((status_hbm.at[pl.ds(nbase, cnt)],
                    status_v.at[pl.ds(0, cnt)]))
        prs.append((area_hbm.at[pl.ds(nbase, cnt)],
                    area_v.at[pl.ds(0, cnt)]))
        return prs

    @pl.when(sid < NS - 1)
    def _():
        for s, d in stage_pairs(STAGE):
            pltpu.async_copy(s, d, sem)

    @pl.when(sid == NS - 1)
    def _():
        for s, d in stage_pairs(STAGE_TAIL):
            pltpu.async_copy(s, d, sem)

    @pl.when(jnp.logical_not(is_tail))
    def _():
        for s, d in chunk_pairs(N_PER_W):
            pltpu.async_copy(s, d, sem)

    @pl.when(is_tail)
    def _():
        for s, d in chunk_pairs(N_TAIL):
            pltpu.async_copy(s, d, sem)

        def zfill(i, carry):
            j, c = i // (CHUNKS - TAIL_CHUNKS), i % (CHUNKS - TAIL_CHUNKS)
            idx_v[pl.ds(j * N_PER_W + N_TAIL + c * LANES, LANES)] = (
                jnp.zeros((LANES,), jnp.int32))
            return carry
        lax.fori_loop(0, K * (CHUNKS - TAIL_CHUNKS), zfill, 0)

    @pl.when(sid < NS - 1)
    def _():
        for s, d in stage_pairs(STAGE):
            pltpu.make_async_copy(s, d, sem).wait()

    @pl.when(sid == NS - 1)
    def _():
        for s, d in stage_pairs(STAGE_TAIL):
            pltpu.make_async_copy(s, d, sem).wait()

    @pl.when(jnp.logical_not(is_tail))
    def _():
        for s, d in chunk_pairs(N_PER_W):
            pltpu.make_async_copy(s, d, sem).wait()

    @pl.when(is_tail)
    def _():
        for s, d in chunk_pairs(N_TAIL):
            pltpu.make_async_copy(s, d, sem).wait()

    @pl.when(sid < NS - 1)
    def _():
        pltpu.sync_copy(gath_v.at[pl.ds(0, STAGE)],
                        table_sp.at[pl.ds(stage_off, STAGE)])

    @pl.when(sid == NS - 1)
    def _():
        pltpu.sync_copy(gath_v.at[pl.ds(0, STAGE_TAIL)],
                        table_sp.at[pl.ds((NS - 1) * STAGE, STAGE_TAIL)])

    plsc.subcore_barrier()

    pltpu.async_copy(table_sp.at[idx_v], gath_v, sem).wait()

    def chunk(c, carry):
        off = c * LANES
        acc = jnp.zeros((LANES,), jnp.float32)
        for j in range(K):
            g = gath_v[pl.ds(j * N_PER_W + off, LANES)]
            d = dirs_v[pl.ds(j * N_PER_W + off, LANES)]
            acc = acc + d * g
        st = status_v[pl.ds(off, LANES)]
        ar = area_v[pl.ds(off, LANES)]
        out_v[pl.ds(off, LANES)] = jnp.where(st == 0, acc / ar, 0.0)
        return carry

    lax.fori_loop(0, CHUNKS, chunk, 0)

    @pl.when(jnp.logical_not(is_tail))
    def _():
        pltpu.sync_copy(out_v, out_hbm.at[pl.ds(nbase, N_PER_W)])

    @pl.when(is_tail)
    def _():
        pltpu.sync_copy(out_v.at[pl.ds(0, N_TAIL)],
                        out_hbm.at[pl.ds(nbase, N_TAIL)])


@jax.jit
def _flux_div_sc(array, links_T, dirs_T, status, area):
    mesh = plsc.VectorSubcoreMesh(core_axis_name="c", subcore_axis_name="s")
    run = pl.kernel(
        _sc_body,
        out_type=jax.ShapeDtypeStruct((N_NODES,), jnp.float32),
        mesh=mesh,
        scratch_types=[
            pltpu.VMEM_SHARED((N_LINKS,), jnp.float32),
            pltpu.VMEM((IDX_PER_W,), jnp.int32),
            pltpu.VMEM((IDX_PER_W,), jnp.float32),
            pltpu.VMEM((IDX_PER_W,), jnp.float32),
            pltpu.VMEM((N_PER_W,), jnp.int32),
            pltpu.VMEM((N_PER_W,), jnp.float32),
            pltpu.VMEM((N_PER_W,), jnp.float32),
            pltpu.SemaphoreType.DMA,
        ],
        compiler_params=pltpu.CompilerParams(needs_layout_passes=False),
    )
    return run(array, links_T, dirs_T, status, area)


def kernel(array, cell_area_at_node, links_at_node, link_dirs_at_node, status_at_node):
    return _flux_div_sc(
        array,
        jnp.swapaxes(links_at_node, 0, 1).reshape(K * N_NODES),
        jnp.swapaxes(link_dirs_at_node, 0, 1).astype(jnp.float32).reshape(K * N_NODES),
        status_at_node,
        cell_area_at_node,
    )
